# Initial kernel scaffold; baseline (speedup 1.0000x reference)
import jax
import jax.numpy as jnp
from jax.experimental import pallas as pl


def kernel(x, W1, g1, b1, W2, g2, b2, W3, g3, b3, W4, g4, b4, W5, g5, b5):
    raise NotImplementedError("write your pallas kernel here")


# speedup vs baseline: 10.5022x; 10.5022x over previous
import functools

import jax
import jax.numpy as jnp
from jax.experimental import pallas as pl

KNN = 20
EPS = 1e-5
NEG = -3e38
BIG = 3e38


def _edge_body(xn_ref, w_ref, mx_ref, s1_ref, s2_ref, *, n, din):
    x = xn_ref[0]
    w16 = w_ref[...]
    xb = x.astype(jnp.bfloat16)
    inner = -2.0 * jax.lax.dot_general(xb, xb, (((1,), (1,)), ((), ())),
                                       preferred_element_type=jnp.float32)
    xsq = jnp.sum(x * x, axis=1)
    dist = (xsq[:, None] + inner) + xsq[None, :]
    col = jax.lax.broadcasted_iota(jnp.int32, (n, n), 1).astype(jnp.float32)
    o = w16.shape[1]
    dp = x.shape[1]
    xc = x[:, :din]
    pad2 = w_ref.shape[0] - 2 * din
    hi = x.astype(jnp.bfloat16)
    hif = hi.astype(jnp.float32)
    mid = (x - hif).astype(jnp.bfloat16)
    midf = mid.astype(jnp.float32)
    lo = (x - hif - midf).astype(jnp.bfloat16)
    splits = jnp.concatenate([hi, mid, lo], axis=1)

    def step(_, carry):
        g_mat, mxa, sa, qa = carry
        rmin = jnp.min(g_mat, axis=1, keepdims=True)
        cand = jnp.where(g_mat == rmin, col, BIG)
        amin = jnp.min(cand, axis=1, keepdims=True)
        onehot = col == amin
        p = jnp.dot(onehot.astype(jnp.bfloat16), splits,
                    preferred_element_type=jnp.float32)
        xg = (p[:, :dp] + p[:, dp:2 * dp]) + p[:, 2 * dp:]
        parts = [xc, xg[:, :din] - xc]
        if pad2:
            parts.append(jnp.zeros((n, pad2), jnp.float32))
        ef16 = jnp.concatenate(parts, axis=1).astype(jnp.bfloat16)
        z = jnp.dot(ef16, w16, preferred_element_type=jnp.float32)
        g_mat = jnp.where(onehot, BIG, g_mat)
        return g_mat, jnp.maximum(mxa, z), sa + z, qa + z * z

    init = (dist,
            jnp.full((n, o), NEG, jnp.float32),
            jnp.zeros((n, o), jnp.float32),
            jnp.zeros((n, o), jnp.float32))
    _, mxa, sa, qa = jax.lax.fori_loop(0, KNN, step, init)
    mx_ref[0] = mxa
    s1_ref[0, 0] = jnp.sum(sa, axis=0)
    s2_ref[0, 0] = jnp.sum(qa, axis=0)


def _edge_layer(xn, w16, din):
    b, n, d = xn.shape
    tw, o = w16.shape
    return pl.pallas_call(
        functools.partial(_edge_body, n=n, din=din),
        grid=(b,),
        in_specs=[pl.BlockSpec((1, n, d), lambda i: (i, 0, 0)),
                  pl.BlockSpec((tw, o), lambda i: (0, 0))],
        out_specs=[pl.BlockSpec((1, n, o), lambda i: (i, 0, 0)),
                   pl.BlockSpec((1, 1, o), lambda i: (i, 0, 0)),
                   pl.BlockSpec((1, 1, o), lambda i: (i, 0, 0))],
        out_shape=[jax.ShapeDtypeStruct((b, n, o), jnp.float32),
                   jax.ShapeDtypeStruct((b, 1, o), jnp.float32),
                   jax.ShapeDtypeStruct((b, 1, o), jnp.float32)],
    )(xn, w16)


def _finalize_body(mx_ref, s1_ref, s2_ref, g_ref, b_ref, out_ref, *, cnt):
    s1 = jnp.sum(s1_ref[...], axis=(0, 1))
    s2 = jnp.sum(s2_ref[...], axis=(0, 1))
    mean = s1 / cnt
    var = s2 / cnt - mean * mean
    sd = jnp.sqrt(var + EPS)
    u = (mx_ref[...] - mean[None, None, :]) / sd[None, None, :]
    v = u * g_ref[0, 0][None, None, :] + b_ref[0, 0][None, None, :]
    out_ref[...] = jnp.where(v >= 0, v, 0.2 * v)


def _finalize(mx, s1, s2, g, b, cnt):
    return pl.pallas_call(
        functools.partial(_finalize_body, cnt=float(cnt)),
        out_shape=jax.ShapeDtypeStruct(mx.shape, jnp.float32),
    )(mx, s1, s2, g.reshape(1, 1, -1), b.reshape(1, 1, -1))


def _layer5_body(cat_ref, w_ref, mx_ref, s1_ref, s2_ref):
    z = jnp.dot(cat_ref[0].astype(jnp.bfloat16), w_ref[...],
                preferred_element_type=jnp.float32)
    mx_ref[0, 0] = jnp.max(z, axis=0)
    s1_ref[0, 0] = jnp.sum(z, axis=0)
    s2_ref[0, 0] = jnp.sum(z * z, axis=0)


def _layer5(cat, w16):
    b, n, d = cat.shape
    o = w16.shape[1]
    return pl.pallas_call(
        _layer5_body,
        grid=(b,),
        in_specs=[pl.BlockSpec((1, n, d), lambda i: (i, 0, 0)),
                  pl.BlockSpec((d, o), lambda i: (0, 0))],
        out_specs=[pl.BlockSpec((1, 1, o), lambda i: (i, 0, 0)),
                   pl.BlockSpec((1, 1, o), lambda i: (i, 0, 0)),
                   pl.BlockSpec((1, 1, o), lambda i: (i, 0, 0))],
        out_shape=[jax.ShapeDtypeStruct((b, 1, o), jnp.float32),
                   jax.ShapeDtypeStruct((b, 1, o), jnp.float32),
                   jax.ShapeDtypeStruct((b, 1, o), jnp.float32)],
    )(cat, w16)


def kernel(x, W1, g1, b1, W2, g2, b2, W3, g3, b3, W4, g4, b4, W5, g5, b5):
    b, _, n = x.shape
    xt = jnp.swapaxes(x, 2, 1)
    xt = jnp.pad(xt, ((0, 0), (0, 0), (0, 5)))

    feats = []
    cur = xt
    for (w, g, bb) in ((W1, g1, b1), (W2, g2, b2), (W3, g3, b3), (W4, g4, b4)):
        din = w.shape[1] // 2
        wc, wn = w[:, :din], w[:, din:]
        w2 = jnp.concatenate([wc.T, wn.T], axis=0)
        if din < 8:
            w2 = jnp.pad(w2, ((0, 16 - 2 * din), (0, 0)))
        mx, s1, s2 = _edge_layer(cur, w2.astype(jnp.bfloat16), din)
        cur = _finalize(mx, s1, s2, g, bb, b * n * KNN)
        feats.append(cur)

    cat = jnp.concatenate(feats, axis=2)
    mxn, s1, s2 = _layer5(cat, W5.T.astype(jnp.bfloat16))
    out = _finalize(mxn, s1, s2, g5, b5, b * n)
    return out.reshape(b, -1)
